# baseline (device time: 163361 ns/iter reference)
import functools

import jax
import jax.numpy as jnp
from jax import lax
from jax.experimental import pallas as pl
from jax.experimental.pallas import tpu as pltpu

N_DEV = 16
M = 1024
CHUNK = M // N_DEV


def kernel(x, W1, W2):
    m, k = x.shape
    _, h_per = W1.shape
    _, n = W2.shape

    def body(x_ref, w1_ref, w2_ref, out_ref, rs_buf, ag_buf,
             send_sem, rs_sems, ag_sems):
        my = lax.axis_index("i")
        left = lax.rem(my + N_DEV - 1, N_DEV)
        right = lax.rem(my + 1, N_DEV)

        barrier_sem = pltpu.get_barrier_semaphore()
        for nbr in (left, right):
            pl.semaphore_signal(
                barrier_sem, inc=1,
                device_id=(nbr,), device_id_type=pl.DeviceIdType.MESH,
            )
        pl.semaphore_wait(barrier_sem, 2)

        xb = x_ref[...].astype(jnp.bfloat16)
        w1b = w1_ref[...].astype(jnp.bfloat16)
        w2b = w2_ref[...].astype(jnp.bfloat16)
        h = jnp.dot(xb, w1b, preferred_element_type=jnp.float32)
        hb = jnp.maximum(h, 0.0).astype(jnp.bfloat16)
        out_ref[...] = jnp.dot(hb, w2b, preferred_element_type=jnp.float32)

        def rows(c):
            return (pl.ds(c * CHUNK, CHUNK), slice(None))

        for s in range(N_DEV - 1):
            c_send = lax.rem(my + N_DEV - s, N_DEV)
            c_recv = lax.rem(my + 2 * N_DEV - 1 - s, N_DEV)
            send = pltpu.make_async_remote_copy(
                src_ref=out_ref.at[rows(c_send)],
                dst_ref=rs_buf.at[c_send],
                send_sem=send_sem,
                recv_sem=rs_sems.at[c_send],
                device_id=(right,),
                device_id_type=pl.DeviceIdType.MESH,
            )
            send.start()
            recv = pltpu.make_async_remote_copy(
                src_ref=rs_buf.at[c_recv],
                dst_ref=rs_buf.at[c_recv],
                send_sem=send_sem,
                recv_sem=rs_sems.at[c_recv],
                device_id=(left,),
                device_id_type=pl.DeviceIdType.MESH,
            )
            recv.wait_recv()
            out_ref[rows(c_recv)] = out_ref[rows(c_recv)] + rs_buf[c_recv]
            send.wait_send()

        for s in range(N_DEV - 1):
            c_send = lax.rem(my + N_DEV + 1 - s, N_DEV)
            c_recv = lax.rem(my + N_DEV - s, N_DEV)
            send = pltpu.make_async_remote_copy(
                src_ref=out_ref.at[rows(c_send)],
                dst_ref=ag_buf.at[c_send],
                send_sem=send_sem,
                recv_sem=ag_sems.at[c_send],
                device_id=(right,),
                device_id_type=pl.DeviceIdType.MESH,
            )
            send.start()
            recv = pltpu.make_async_remote_copy(
                src_ref=ag_buf.at[c_recv],
                dst_ref=ag_buf.at[c_recv],
                send_sem=send_sem,
                recv_sem=ag_sems.at[c_recv],
                device_id=(left,),
                device_id_type=pl.DeviceIdType.MESH,
            )
            recv.wait_recv()
            out_ref[rows(c_recv)] = ag_buf[c_recv]
            send.wait_send()

    return pl.pallas_call(
        body,
        out_shape=jax.ShapeDtypeStruct((m, n), jnp.float32),
        in_specs=[
            pl.BlockSpec(memory_space=pltpu.VMEM),
            pl.BlockSpec(memory_space=pltpu.VMEM),
            pl.BlockSpec(memory_space=pltpu.VMEM),
        ],
        out_specs=pl.BlockSpec(memory_space=pltpu.VMEM),
        scratch_shapes=[
            pltpu.VMEM((N_DEV, CHUNK, n), jnp.float32),
            pltpu.VMEM((N_DEV, CHUNK, n), jnp.float32),
            pltpu.SemaphoreType.DMA,
            pltpu.SemaphoreType.DMA((N_DEV,)),
            pltpu.SemaphoreType.DMA((N_DEV,)),
        ],
        compiler_params=pltpu.CompilerParams(collective_id=0),
    )(x, W1, W2)


# device time: 73452 ns/iter; 2.2241x vs baseline; 2.2241x over previous
import functools

import jax
import jax.numpy as jnp
from jax import lax
from jax.experimental import pallas as pl
from jax.experimental.pallas import tpu as pltpu

N_DEV = 16
M = 1024
CHUNK = M // N_DEV


def kernel(x, W1, W2):
    m, k = x.shape
    _, h_per = W1.shape
    _, n = W2.shape

    def body(x_ref, w1_ref, w2_ref, out_ref,
             stage, rs_buf, ag_stage, ag_buf, send_sems, recv_sems):
        my = lax.axis_index("i")

        def peers():
            for o in range(1, N_DEV):
                yield lax.rem(my + o, N_DEV)

        barrier_sem = pltpu.get_barrier_semaphore()
        for p in peers():
            pl.semaphore_signal(
                barrier_sem, inc=1,
                device_id=(p,), device_id_type=pl.DeviceIdType.MESH,
            )
        pl.semaphore_wait(barrier_sem, N_DEV - 1)

        xb = x_ref[...].astype(jnp.bfloat16)
        w1b = w1_ref[...].astype(jnp.bfloat16)
        w2b = w2_ref[...].astype(jnp.bfloat16)
        h = jnp.dot(xb, w1b, preferred_element_type=jnp.float32)
        hb = jnp.maximum(h, 0.0).astype(jnp.bfloat16)
        partial = jnp.dot(hb, w2b, preferred_element_type=jnp.float32)
        stage[...] = partial.astype(jnp.bfloat16).reshape(N_DEV, CHUNK, n)

        for p in peers():
            pltpu.make_async_remote_copy(
                src_ref=stage.at[p],
                dst_ref=rs_buf.at[my],
                send_sem=send_sems.at[p],
                recv_sem=recv_sems.at[my],
                device_id=(p,),
                device_id_type=pl.DeviceIdType.MESH,
            ).start()
        rs_buf[my] = stage[my]
        for p in peers():
            pltpu.make_async_remote_copy(
                src_ref=rs_buf.at[p], dst_ref=rs_buf.at[p],
                send_sem=send_sems.at[p], recv_sem=recv_sems.at[p],
                device_id=(p,), device_id_type=pl.DeviceIdType.MESH,
            ).wait_recv()
        for p in peers():
            pltpu.make_async_remote_copy(
                src_ref=stage.at[p], dst_ref=rs_buf.at[my],
                send_sem=send_sems.at[p], recv_sem=recv_sems.at[my],
                device_id=(p,), device_id_type=pl.DeviceIdType.MESH,
            ).wait_send()
        red = jnp.sum(rs_buf[...].astype(jnp.float32), axis=0)
        ag_stage[...] = red.astype(jnp.bfloat16)

        @functools.partial(
            pl.run_scoped, mid_barrier=pltpu.SemaphoreType.REGULAR
        )
        def _(mid_barrier):
            for p in peers():
                pl.semaphore_signal(
                    mid_barrier, inc=1,
                    device_id=(p,), device_id_type=pl.DeviceIdType.MESH,
                )
            pl.semaphore_wait(mid_barrier, N_DEV - 1)

        for p in peers():
            pltpu.make_async_remote_copy(
                src_ref=ag_stage,
                dst_ref=ag_buf.at[my],
                send_sem=send_sems.at[p],
                recv_sem=recv_sems.at[my],
                device_id=(p,),
                device_id_type=pl.DeviceIdType.MESH,
            ).start()
        ag_buf[my] = ag_stage[...]
        for p in peers():
            pltpu.make_async_remote_copy(
                src_ref=ag_buf.at[p], dst_ref=ag_buf.at[p],
                send_sem=send_sems.at[p], recv_sem=recv_sems.at[p],
                device_id=(p,), device_id_type=pl.DeviceIdType.MESH,
            ).wait_recv()
        out_ref[...] = ag_buf[...].reshape(m, n)
        for p in peers():
            pltpu.make_async_remote_copy(
                src_ref=ag_stage, dst_ref=ag_buf.at[my],
                send_sem=send_sems.at[p], recv_sem=recv_sems.at[my],
                device_id=(p,), device_id_type=pl.DeviceIdType.MESH,
            ).wait_send()

    return pl.pallas_call(
        body,
        out_shape=jax.ShapeDtypeStruct((m, n), jnp.bfloat16),
        in_specs=[
            pl.BlockSpec(memory_space=pltpu.VMEM),
            pl.BlockSpec(memory_space=pltpu.VMEM),
            pl.BlockSpec(memory_space=pltpu.VMEM),
        ],
        out_specs=pl.BlockSpec(memory_space=pltpu.VMEM),
        scratch_shapes=[
            pltpu.VMEM((N_DEV, CHUNK, n), jnp.bfloat16),
            pltpu.VMEM((N_DEV, CHUNK, n), jnp.bfloat16),
            pltpu.VMEM((CHUNK, n), jnp.bfloat16),
            pltpu.VMEM((N_DEV, CHUNK, n), jnp.bfloat16),
            pltpu.SemaphoreType.DMA((N_DEV,)),
            pltpu.SemaphoreType.DMA((N_DEV,)),
        ],
        compiler_params=pltpu.CompilerParams(collective_id=0),
    )(x, W1, W2)


# device time: 65117 ns/iter; 2.5087x vs baseline; 1.1280x over previous
import functools

import jax
import jax.numpy as jnp
from jax import lax
from jax.experimental import pallas as pl
from jax.experimental.pallas import tpu as pltpu

N_DEV = 16
M = 1024
CHUNK = M // N_DEV
NQ = 4
QROWS = M // NQ


def kernel(x, W1, W2):
    m, k = x.shape
    _, h_per = W1.shape
    _, n = W2.shape

    def body(x_ref, w1_ref, w2_ref, out_ref,
             stage, rs_buf, ag_stage, send_sems, recv_sems):
        my = lax.axis_index("i")

        def peers():
            for o in range(1, N_DEV):
                yield lax.rem(my + o, N_DEV)

        barrier_sem = pltpu.get_barrier_semaphore()
        for p in peers():
            pl.semaphore_signal(
                barrier_sem, inc=1,
                device_id=(p,), device_id_type=pl.DeviceIdType.MESH,
            )
        pl.semaphore_wait(barrier_sem, N_DEV - 1)

        w1b = w1_ref[...].astype(jnp.bfloat16)
        w2b = w2_ref[...].astype(jnp.bfloat16)

        for i in range(NQ):
            q = lax.rem(my // NQ + 1 + i, NQ)
            xq = x_ref[pl.ds(q * QROWS, QROWS), :].astype(jnp.bfloat16)
            hq = jnp.dot(xq, w1b, preferred_element_type=jnp.float32)
            hqb = jnp.maximum(hq, 0.0).astype(jnp.bfloat16)
            pq = jnp.dot(hqb, w2b, preferred_element_type=jnp.float32)
            stage[pl.ds(NQ * q, NQ)] = (
                pq.astype(jnp.bfloat16).reshape(NQ, CHUNK, n)
            )
            for j in range(NQ):
                c = NQ * q + lax.rem(my + j, NQ)

                @pl.when(c != my)
                def _(c=c):
                    pltpu.make_async_remote_copy(
                        src_ref=stage.at[c],
                        dst_ref=rs_buf.at[my],
                        send_sem=send_sems.at[c],
                        recv_sem=recv_sems.at[my],
                        device_id=(c,),
                        device_id_type=pl.DeviceIdType.MESH,
                    ).start()

        rs_buf[my] = stage[my]
        for p in peers():
            pltpu.make_async_remote_copy(
                src_ref=rs_buf.at[p], dst_ref=rs_buf.at[p],
                send_sem=send_sems.at[p], recv_sem=recv_sems.at[p],
                device_id=(p,), device_id_type=pl.DeviceIdType.MESH,
            ).wait_recv()
        for p in peers():
            pltpu.make_async_remote_copy(
                src_ref=stage.at[p], dst_ref=rs_buf.at[my],
                send_sem=send_sems.at[p], recv_sem=recv_sems.at[my],
                device_id=(p,), device_id_type=pl.DeviceIdType.MESH,
            ).wait_send()
        red = jnp.sum(rs_buf[...].astype(jnp.float32), axis=0)
        ag_stage[...] = red.astype(jnp.bfloat16)

        @functools.partial(
            pl.run_scoped, mid_barrier=pltpu.SemaphoreType.REGULAR
        )
        def _(mid_barrier):
            for p in peers():
                pl.semaphore_signal(
                    mid_barrier, inc=1,
                    device_id=(p,), device_id_type=pl.DeviceIdType.MESH,
                )
            pl.semaphore_wait(mid_barrier, N_DEV - 1)

        for p in peers():
            pltpu.make_async_remote_copy(
                src_ref=ag_stage,
                dst_ref=out_ref.at[pl.ds(my * CHUNK, CHUNK), :],
                send_sem=send_sems.at[p],
                recv_sem=recv_sems.at[my],
                device_id=(p,),
                device_id_type=pl.DeviceIdType.MESH,
            ).start()
        out_ref[pl.ds(my * CHUNK, CHUNK), :] = ag_stage[...]
        for p in peers():
            pltpu.make_async_remote_copy(
                src_ref=ag_stage,
                dst_ref=out_ref.at[pl.ds(p * CHUNK, CHUNK), :],
                send_sem=send_sems.at[p], recv_sem=recv_sems.at[p],
                device_id=(p,), device_id_type=pl.DeviceIdType.MESH,
            ).wait_recv()
        for p in peers():
            pltpu.make_async_remote_copy(
                src_ref=ag_stage,
                dst_ref=out_ref.at[pl.ds(my * CHUNK, CHUNK), :],
                send_sem=send_sems.at[p], recv_sem=recv_sems.at[my],
                device_id=(p,), device_id_type=pl.DeviceIdType.MESH,
            ).wait_send()

    return pl.pallas_call(
        body,
        out_shape=jax.ShapeDtypeStruct((m, n), jnp.bfloat16),
        in_specs=[
            pl.BlockSpec(memory_space=pltpu.VMEM),
            pl.BlockSpec(memory_space=pltpu.VMEM),
            pl.BlockSpec(memory_space=pltpu.VMEM),
        ],
        out_specs=pl.BlockSpec(memory_space=pltpu.VMEM),
        scratch_shapes=[
            pltpu.VMEM((N_DEV, CHUNK, n), jnp.bfloat16),
            pltpu.VMEM((N_DEV, CHUNK, n), jnp.bfloat16),
            pltpu.VMEM((CHUNK, n), jnp.bfloat16),
            pltpu.SemaphoreType.DMA((N_DEV,)),
            pltpu.SemaphoreType.DMA((N_DEV,)),
        ],
        compiler_params=pltpu.CompilerParams(collective_id=0),
    )(x, W1, W2)


# device time: 61349 ns/iter; 2.6628x vs baseline; 1.0614x over previous
import functools

import jax
import jax.numpy as jnp
from jax import lax
from jax.experimental import pallas as pl
from jax.experimental.pallas import tpu as pltpu

N_DEV = 16
M = 1024
CHUNK = M // N_DEV
NQ = 4
QROWS = M // NQ


def kernel(x, W1, W2):
    m, k = x.shape
    _, h_per = W1.shape
    _, n = W2.shape

    def body(x_ref, w1_ref, w2_ref, out_ref,
             stage, rs_buf, ag_stage, send_sems, recv_sems, ag_sems):
        my = lax.axis_index("i")

        def peers():
            for o in range(1, N_DEV):
                yield lax.rem(my + o, N_DEV)

        barrier_sem = pltpu.get_barrier_semaphore()
        for p in peers():
            pl.semaphore_signal(
                barrier_sem, inc=1,
                device_id=(p,), device_id_type=pl.DeviceIdType.MESH,
            )
        pl.semaphore_wait(barrier_sem, N_DEV - 1)

        w1b = w1_ref[...].astype(jnp.bfloat16)
        w2b = w2_ref[...].astype(jnp.bfloat16)

        for i in range(NQ):
            q = lax.rem(my // NQ + 1 + i, NQ)
            xq = x_ref[pl.ds(q * QROWS, QROWS), :].astype(jnp.bfloat16)
            hq = jnp.dot(xq, w1b, preferred_element_type=jnp.float32)
            hqb = jnp.maximum(hq, 0.0).astype(jnp.bfloat16)
            pq = jnp.dot(hqb, w2b, preferred_element_type=jnp.float32)
            stage[pl.ds(NQ * q, NQ)] = (
                pq.astype(jnp.bfloat16).reshape(NQ, CHUNK, n)
            )
            for j in range(NQ):
                c = NQ * q + lax.rem(my + j, NQ)

                @pl.when(c != my)
                def _(c=c):
                    pltpu.make_async_remote_copy(
                        src_ref=stage.at[c],
                        dst_ref=rs_buf.at[my],
                        send_sem=send_sems.at[c],
                        recv_sem=recv_sems.at[my],
                        device_id=(c,),
                        device_id_type=pl.DeviceIdType.MESH,
                    ).start()

        red = stage[my].astype(jnp.float32)
        for p in peers():
            pltpu.make_async_remote_copy(
                src_ref=rs_buf.at[p], dst_ref=rs_buf.at[p],
                send_sem=send_sems.at[p], recv_sem=recv_sems.at[p],
                device_id=(p,), device_id_type=pl.DeviceIdType.MESH,
            ).wait_recv()
            red = red + rs_buf[p].astype(jnp.float32)
        ag_stage[...] = red.astype(jnp.bfloat16)
        for p in peers():
            pltpu.make_async_remote_copy(
                src_ref=stage.at[p], dst_ref=rs_buf.at[my],
                send_sem=send_sems.at[p], recv_sem=recv_sems.at[my],
                device_id=(p,), device_id_type=pl.DeviceIdType.MESH,
            ).wait_send()

        for p in peers():
            pltpu.make_async_remote_copy(
                src_ref=ag_stage,
                dst_ref=out_ref.at[pl.ds(my * CHUNK, CHUNK), :],
                send_sem=send_sems.at[p],
                recv_sem=ag_sems.at[my],
                device_id=(p,),
                device_id_type=pl.DeviceIdType.MESH,
            ).start()
        out_ref[pl.ds(my * CHUNK, CHUNK), :] = ag_stage[...]
        for p in peers():
            pltpu.make_async_remote_copy(
                src_ref=ag_stage,
                dst_ref=out_ref.at[pl.ds(p * CHUNK, CHUNK), :],
                send_sem=send_sems.at[p], recv_sem=ag_sems.at[p],
                device_id=(p,), device_id_type=pl.DeviceIdType.MESH,
            ).wait_recv()
        for p in peers():
            pltpu.make_async_remote_copy(
                src_ref=ag_stage,
                dst_ref=out_ref.at[pl.ds(my * CHUNK, CHUNK), :],
                send_sem=send_sems.at[p], recv_sem=ag_sems.at[my],
                device_id=(p,), device_id_type=pl.DeviceIdType.MESH,
            ).wait_send()

    return pl.pallas_call(
        body,
        out_shape=jax.ShapeDtypeStruct((m, n), jnp.bfloat16),
        in_specs=[
            pl.BlockSpec(memory_space=pltpu.VMEM),
            pl.BlockSpec(memory_space=pltpu.VMEM),
            pl.BlockSpec(memory_space=pltpu.VMEM),
        ],
        out_specs=pl.BlockSpec(memory_space=pltpu.VMEM),
        scratch_shapes=[
            pltpu.VMEM((N_DEV, CHUNK, n), jnp.bfloat16),
            pltpu.VMEM((N_DEV, CHUNK, n), jnp.bfloat16),
            pltpu.VMEM((CHUNK, n), jnp.bfloat16),
            pltpu.SemaphoreType.DMA((N_DEV,)),
            pltpu.SemaphoreType.DMA((N_DEV,)),
            pltpu.SemaphoreType.DMA((N_DEV,)),
        ],
        compiler_params=pltpu.CompilerParams(collective_id=0),
    )(x, W1, W2)


# device time: 59564 ns/iter; 2.7426x vs baseline; 1.0300x over previous
import functools

import jax
import jax.numpy as jnp
from jax import lax
from jax.experimental import pallas as pl
from jax.experimental.pallas import tpu as pltpu

N_DEV = 16
M = 1024
CHUNK = M // N_DEV
NQ = 4
QROWS = M // NQ


def kernel(x, W1, W2):
    m, k = x.shape
    _, h_per = W1.shape
    _, n = W2.shape

    def body(x_ref, w1_ref, w2_ref, out_ref,
             stage, rs_buf, ag_stage, send_sems, recv_sems, ag_sems):
        my = lax.axis_index("i")

        def peers():
            for o in range(1, N_DEV):
                yield lax.rem(my + o, N_DEV)

        barrier_sem = pltpu.get_barrier_semaphore()
        for p in peers():
            pl.semaphore_signal(
                barrier_sem, inc=1,
                device_id=(p,), device_id_type=pl.DeviceIdType.MESH,
            )

        w1b = w1_ref[...].astype(jnp.bfloat16)
        w2b = w2_ref[...].astype(jnp.bfloat16)

        for i in range(NQ):
            q = lax.rem(my // NQ + 1 + i, NQ)
            xq = x_ref[pl.ds(q * QROWS, QROWS), :].astype(jnp.bfloat16)
            hq = jnp.dot(xq, w1b, preferred_element_type=jnp.float32)
            hqb = jnp.maximum(hq, 0.0).astype(jnp.bfloat16)
            pq = jnp.dot(hqb, w2b, preferred_element_type=jnp.float32)
            stage[pl.ds(NQ * q, NQ)] = (
                pq.astype(jnp.bfloat16).reshape(NQ, CHUNK, n)
            )
            if i == 0:
                pl.semaphore_wait(barrier_sem, N_DEV - 1)
            for j in range(NQ):
                c = NQ * q + lax.rem(my + j, NQ)

                @pl.when(c != my)
                def _(c=c):
                    pltpu.make_async_remote_copy(
                        src_ref=stage.at[c],
                        dst_ref=rs_buf.at[my],
                        send_sem=send_sems.at[c],
                        recv_sem=recv_sems.at[my],
                        device_id=(c,),
                        device_id_type=pl.DeviceIdType.MESH,
                    ).start()

        red = stage[my].astype(jnp.float32)
        for p in peers():
            pltpu.make_async_remote_copy(
                src_ref=rs_buf.at[p], dst_ref=rs_buf.at[p],
                send_sem=send_sems.at[p], recv_sem=recv_sems.at[p],
                device_id=(p,), device_id_type=pl.DeviceIdType.MESH,
            ).wait_recv()
            red = red + rs_buf[p].astype(jnp.float32)
        ag_stage[...] = red.astype(jnp.bfloat16)
        for p in peers():
            pltpu.make_async_remote_copy(
                src_ref=stage.at[p], dst_ref=rs_buf.at[my],
                send_sem=send_sems.at[p], recv_sem=recv_sems.at[my],
                device_id=(p,), device_id_type=pl.DeviceIdType.MESH,
            ).wait_send()

        for p in peers():
            pltpu.make_async_remote_copy(
                src_ref=ag_stage,
                dst_ref=out_ref.at[pl.ds(my * CHUNK, CHUNK), :],
                send_sem=send_sems.at[p],
                recv_sem=ag_sems.at[my],
                device_id=(p,),
                device_id_type=pl.DeviceIdType.MESH,
            ).start()
        out_ref[pl.ds(my * CHUNK, CHUNK), :] = ag_stage[...]
        for p in peers():
            pltpu.make_async_remote_copy(
                src_ref=ag_stage,
                dst_ref=out_ref.at[pl.ds(p * CHUNK, CHUNK), :],
                send_sem=send_sems.at[p], recv_sem=ag_sems.at[p],
                device_id=(p,), device_id_type=pl.DeviceIdType.MESH,
            ).wait_recv()
        for p in peers():
            pltpu.make_async_remote_copy(
                src_ref=ag_stage,
                dst_ref=out_ref.at[pl.ds(my * CHUNK, CHUNK), :],
                send_sem=send_sems.at[p], recv_sem=ag_sems.at[my],
                device_id=(p,), device_id_type=pl.DeviceIdType.MESH,
            ).wait_send()

    return pl.pallas_call(
        body,
        out_shape=jax.ShapeDtypeStruct((m, n), jnp.bfloat16),
        in_specs=[
            pl.BlockSpec(memory_space=pltpu.VMEM),
            pl.BlockSpec(memory_space=pltpu.VMEM),
            pl.BlockSpec(memory_space=pltpu.VMEM),
        ],
        out_specs=pl.BlockSpec(memory_space=pltpu.VMEM),
        scratch_shapes=[
            pltpu.VMEM((N_DEV, CHUNK, n), jnp.bfloat16),
            pltpu.VMEM((N_DEV, CHUNK, n), jnp.bfloat16),
            pltpu.VMEM((CHUNK, n), jnp.bfloat16),
            pltpu.SemaphoreType.DMA((N_DEV,)),
            pltpu.SemaphoreType.DMA((N_DEV,)),
            pltpu.SemaphoreType.DMA((N_DEV,)),
        ],
        compiler_params=pltpu.CompilerParams(collective_id=0),
    )(x, W1, W2)
